# Initial kernel scaffold; baseline (speedup 1.0000x reference)
import jax
import jax.numpy as jnp
from jax.experimental import pallas as pl


def kernel(expanded_features):
    raise NotImplementedError("write your pallas kernel here")


# speedup vs baseline: 26.1951x; 26.1951x over previous
import functools

import jax
import jax.numpy as jnp
from jax import lax
from jax.experimental import pallas as pl

_KEEP_RATIO = 0.05
_INT_MIN = -(2 ** 31)


def _wta_block(x_ref, o_ref, *, k: int):
    x = x_ref[...]
    s = lax.bitcast_convert_type(x, jnp.int32)
    key = s ^ ((s >> 31) & jnp.int32(0x7FFFFFFF))

    rows = x.shape[0]
    tu = jnp.zeros((rows, 1), jnp.int32)
    kk = jnp.int32(k)
    for b in range(31, -1, -1):
        bit = jnp.int32(_INT_MIN) if b == 31 else jnp.int32(1 << b)
        cu = tu | bit
        cs = cu ^ jnp.int32(_INT_MIN)
        cnt = jnp.sum((key >= cs).astype(jnp.int32), axis=1, keepdims=True)
        tu = jnp.where(cnt >= kk, cu, tu)
    thr = tu ^ jnp.int32(_INT_MIN)

    o_ref[...] = jnp.where(key >= thr, x, jnp.float32(0.0))


@jax.jit
def kernel(expanded_features):
    B, D = expanded_features.shape
    k = max(1, int(D * _KEEP_RATIO))
    block_rows = 16
    grid = (B // block_rows,)
    return pl.pallas_call(
        functools.partial(_wta_block, k=k),
        grid=grid,
        in_specs=[pl.BlockSpec((block_rows, D), lambda i: (i, 0))],
        out_specs=pl.BlockSpec((block_rows, D), lambda i: (i, 0)),
        out_shape=jax.ShapeDtypeStruct((B, D), jnp.float32),
    )(expanded_features)
